# baseline (speedup 1.0000x reference)
import functools

import jax
import jax.numpy as jnp
from jax import lax
from jax.experimental import pallas as pl
from jax.experimental.pallas import tpu as pltpu
from jax.experimental.pallas import tpu_sc as plsc

NA, NBND, NMOL = 50000, 100000, 4096
AF, BF, HID = 133, 147, 300

HP = 384
BFP = 152
AFP = 136
NAP = 50176
NBP = 100352
PBW = 3136
PBW16 = 6272
NSB16 = 98
GB = 64
NGB = 49
CH = 16
NCH = 24
NCHS = 19
SBB = 112
NSBB = 56
BIG = 784
NBIG = 8
NMP = 4352
APW16 = 3136

_f32 = jnp.float32


def _bf(x):
    return x.astype(jnp.bfloat16)


def _mm_in_body(fb_ref, wi_ref, wh_ref, inp_ref, zp_ref, zn_ref):
    acc = jnp.dot(_bf(fb_ref[...]), _bf(wi_ref[...]),
                  preferred_element_type=_f32)
    inp_ref[...] = acc
    z = jnp.dot(_bf(jnp.maximum(acc, 0.0)), _bf(wh_ref[...]),
                preferred_element_type=_f32)
    zp_ref[...] = z
    zn_ref[...] = -z


def _mm_in(fb_p, wi_p, wh_p):
    blk = 1024
    return pl.pallas_call(
        _mm_in_body,
        grid=(NBP // blk,),
        in_specs=[
            pl.BlockSpec((blk, BFP), lambda i: (i, 0)),
            pl.BlockSpec((BFP, HP), lambda i: (0, 0)),
            pl.BlockSpec((HP, HP), lambda i: (0, 0)),
        ],
        out_specs=[
            pl.BlockSpec((blk, HP), lambda i: (i, 0)),
            pl.BlockSpec((blk, HP), lambda i: (i, 0)),
            pl.BlockSpec((blk, HP), lambda i: (i, 0)),
        ],
        out_shape=[
            jax.ShapeDtypeStruct((NBP, HP), _f32),
            jax.ShapeDtypeStruct((NBP, HP), _f32),
            jax.ShapeDtypeStruct((NBP, HP), _f32),
        ],
    )(fb_p, wi_p, wh_p)


def _mm_update_body(g1_ref, g2_ref, inp_ref, wh_ref, zp_ref, zn_ref):
    m = jnp.maximum(inp_ref[...] + g1_ref[...] + g2_ref[...], 0.0)
    z = jnp.dot(_bf(m), _bf(wh_ref[...]), preferred_element_type=_f32)
    zp_ref[...] = z
    zn_ref[...] = -z


def _mm_update(g1, g2, inp, wh_p):
    blk = 1024
    return pl.pallas_call(
        _mm_update_body,
        grid=(NBP // blk,),
        in_specs=[
            pl.BlockSpec((blk, HP), lambda i: (i, 0)),
            pl.BlockSpec((blk, HP), lambda i: (i, 0)),
            pl.BlockSpec((blk, HP), lambda i: (i, 0)),
            pl.BlockSpec((HP, HP), lambda i: (0, 0)),
        ],
        out_specs=[
            pl.BlockSpec((blk, HP), lambda i: (i, 0)),
            pl.BlockSpec((blk, HP), lambda i: (i, 0)),
        ],
        out_shape=[
            jax.ShapeDtypeStruct((NBP, HP), _f32),
            jax.ShapeDtypeStruct((NBP, HP), _f32),
        ],
    )(g1, g2, inp, wh_p)


def _mm_update_last_body(g1_ref, g2_ref, inp_ref, wo2_ref, y_ref):
    m = jnp.maximum(inp_ref[...] + g1_ref[...] + g2_ref[...], 0.0)
    y_ref[...] = jnp.dot(_bf(m), _bf(wo2_ref[...]), preferred_element_type=_f32)


def _mm_update_last(g1, g2, inp, wo2_p):
    blk = 1024
    return pl.pallas_call(
        _mm_update_last_body,
        grid=(NBP // blk,),
        in_specs=[
            pl.BlockSpec((blk, HP), lambda i: (i, 0)),
            pl.BlockSpec((blk, HP), lambda i: (i, 0)),
            pl.BlockSpec((blk, HP), lambda i: (i, 0)),
            pl.BlockSpec((HP, HP), lambda i: (0, 0)),
        ],
        out_specs=pl.BlockSpec((blk, HP), lambda i: (i, 0)),
        out_shape=jax.ShapeDtypeStruct((NBP, HP), _f32),
    )(g1, g2, inp, wo2_p)


def _mm_out_body(fa_ref, ya_ref, wo1_ref, bo_ref, wid_ref, s_ref):
    h = (jnp.dot(_bf(fa_ref[...]), _bf(wo1_ref[...]),
                 preferred_element_type=_f32)
         + ya_ref[...] + bo_ref[...])
    h = jnp.maximum(h, 0.0)
    s_ref[...] = jnp.dot(_bf(h), _bf(wid_ref[...]), preferred_element_type=_f32)


def _mm_out(fa_p, ya, wo1_p, bo_p, wid_p):
    blk = 1024
    return pl.pallas_call(
        _mm_out_body,
        grid=(NAP // blk,),
        in_specs=[
            pl.BlockSpec((blk, AFP), lambda i: (i, 0)),
            pl.BlockSpec((blk, HP), lambda i: (i, 0)),
            pl.BlockSpec((AFP, HP), lambda i: (0, 0)),
            pl.BlockSpec((1, HP), lambda i: (0, 0)),
            pl.BlockSpec((HP, 128), lambda i: (0, 0)),
        ],
        out_specs=pl.BlockSpec((blk, 128), lambda i: (i, 0)),
        out_shape=jax.ShapeDtypeStruct((NAP, 128), _f32),
    )(fa_p, ya, wo1_p, bo_p, wid_p)


_SC_MESH = dict(core_axis_name="c", subcore_axis_name="s")


def _scatter_body(zc_hbm, dst_hbm, am_hbm, idx_v, vals_v, zero_v, table,
                  lsem, ssem):
    cid = lax.axis_index("c")
    sid = lax.axis_index("s")
    base = sid * PBW16
    pltpu.sync_copy(dst_hbm.at[sid], idx_v)

    def _zrow(i, carry):
        zero_v[i, :] = jnp.zeros((16,), _f32)
        return carry

    lax.fori_loop(0, 392, _zrow, 0)

    for c2 in range(NCHS, NCH):
        @pl.when(cid == (c2 % 2))
        def _zc(c2=c2):
            for z in range(8):
                pltpu.sync_copy(
                    zero_v, am_hbm.at[pl.ds(sid * 3136 + z * 392, 392), c2, :])

    def _chunk(c, carry):
        ch_id = cid * 10 + c
        for z in range(8):
            pltpu.sync_copy(zero_v, table.at[pl.ds(sid * 3136 + z * 392, 392)])
        plsc.subcore_barrier()
        ld = [None, None]
        pend = [[], []]
        ld[0] = pltpu.async_copy(
            zc_hbm.at[pl.ds(base, BIG), ch_id, :], vals_v.at[0], lsem)
        for b in range(NBIG):
            s = b & 1
            ld[s].wait()
            cur = []
            for k in range(7):
                cur.append(pltpu.async_copy(
                    vals_v.at[s, pl.ds(k * SBB, SBB), :],
                    table.at[idx_v.at[b * 7 + k]], ssem, add=True))
            for d in pend[1 - s]:
                d.wait()
            pend[1 - s] = []
            if b + 1 < NBIG:
                ld[1 - s] = pltpu.async_copy(
                    zc_hbm.at[pl.ds(base + (b + 1) * BIG, BIG), ch_id, :],
                    vals_v.at[1 - s], lsem)
            pend[s] = cur
        for d in pend[0] + pend[1]:
            d.wait()
        plsc.subcore_barrier()
        pltpu.sync_copy(
            table.at[pl.ds(sid * 3136, 3136)],
            am_hbm.at[pl.ds(sid * 3136, 3136), ch_id, :])
        plsc.subcore_barrier()
        return carry

    lax.fori_loop(0, 10 - cid, _chunk, 0)


def _scatter(z, dst3):
    zc = z[:, :NCHS * CH].reshape(NBP, NCHS, CH)
    am_c = pl.kernel(
        _scatter_body,
        out_type=jax.ShapeDtypeStruct((NAP, NCH, CH), _f32),
        mesh=plsc.VectorSubcoreMesh(**_SC_MESH),
        scratch_types=[
            pltpu.VMEM((NSBB, SBB), jnp.int32),
            pltpu.VMEM((2, BIG, CH), _f32),
            pltpu.VMEM((392, CH), _f32),
            pltpu.VMEM_SHARED((NAP, CH), _f32),
            pltpu.SemaphoreType.DMA,
            pltpu.SemaphoreType.DMA,
        ],
        compiler_params=pltpu.CompilerParams(use_tc_tiling_on_sc=False),
    )(zc, dst3)
    return am_c.reshape(NAP, HP)


def _gather_body(am_hbm, zn_hbm, b2a_hbm, rev_hbm, g1_hbm, g2_hbm,
                 idxa_v, idxr_v, bufa, bufb, sema, semb, semw):
    cid = lax.axis_index("c")
    sid = lax.axis_index("s")
    w = cid * 16 + sid
    base = w * PBW
    pltpu.sync_copy(b2a_hbm.at[w], idxa_v)
    pltpu.sync_copy(rev_hbm.at[w], idxr_v)
    g = [None, None]
    wb = [[], []]
    g[0] = (pltpu.async_copy(am_hbm.at[idxa_v.at[0]], bufa.at[0], sema),
            pltpu.async_copy(zn_hbm.at[idxr_v.at[0]], bufb.at[0], semb))
    for j in range(NGB):
        s = j & 1
        for d in wb[1 - s]:
            d.wait()
        wb[1 - s] = []
        if j + 1 < NGB:
            g[1 - s] = (
                pltpu.async_copy(am_hbm.at[idxa_v.at[j + 1]], bufa.at[1 - s],
                                 sema),
                pltpu.async_copy(zn_hbm.at[idxr_v.at[j + 1]], bufb.at[1 - s],
                                 semb))
        da, db = g[s]
        da.wait()
        db.wait()
        wb[s] = [
            pltpu.async_copy(bufa.at[s], g1_hbm.at[pl.ds(base + j * GB, GB)],
                             semw),
            pltpu.async_copy(bufb.at[s], g2_hbm.at[pl.ds(base + j * GB, GB)],
                             semw),
        ]
    for d in wb[0] + wb[1]:
        d.wait()


def _gather(am, zn, b2a3, rev3):
    return pl.kernel(
        _gather_body,
        out_type=[jax.ShapeDtypeStruct((NBP, HP), _f32),
                  jax.ShapeDtypeStruct((NBP, HP), _f32)],
        mesh=plsc.VectorSubcoreMesh(**_SC_MESH),
        scratch_types=[
            pltpu.VMEM((NGB, GB), jnp.int32),
            pltpu.VMEM((NGB, GB), jnp.int32),
            pltpu.VMEM((2, GB, HP), _f32),
            pltpu.VMEM((2, GB, HP), _f32),
            pltpu.SemaphoreType.DMA,
            pltpu.SemaphoreType.DMA,
            pltpu.SemaphoreType.DMA,
        ],
    )(am, zn, b2a3, rev3)


def _readout_body(s_hbm, ids_hbm, bvec_hbm, out_hbm,
                  idx_v, vals_v, ones_v, a_v, c_v, o_v, bvec_v, sums, counts):
    cid = lax.axis_index("c")
    sid = lax.axis_index("s")

    @pl.when(cid == 0)
    def _():
        pltpu.sync_copy(ids_hbm.at[sid], idx_v)
        pltpu.sync_copy(s_hbm.at[sid], vals_v)
        for k in range(GB // 16):
            ones_v[pl.ds(k * 16, 16)] = jnp.full((16,), 1.0, _f32)
        for k in range(272 // 16):
            o_v[pl.ds(k * 16, 16)] = jnp.zeros((16,), _f32)
        pltpu.sync_copy(o_v, sums.at[pl.ds(sid * 272, 272)])
        pltpu.sync_copy(o_v, counts.at[pl.ds(sid * 272, 272)])
        plsc.subcore_barrier()
        for j in range(NGB):
            pltpu.sync_copy(vals_v.at[j], sums.at[idx_v.at[j]], add=True)
            pltpu.sync_copy(ones_v, counts.at[idx_v.at[j]], add=True)
        plsc.subcore_barrier()
        pltpu.sync_copy(bvec_hbm, bvec_v)
        pltpu.sync_copy(sums.at[pl.ds(sid * 272, 272)], a_v)
        pltpu.sync_copy(counts.at[pl.ds(sid * 272, 272)], c_v)
        b = bvec_v[...]
        for k in range(272 // 16):
            x = a_v[pl.ds(k * 16, 16)]
            cc = c_v[pl.ds(k * 16, 16)]
            o_v[pl.ds(k * 16, 16)] = x / jnp.maximum(cc, 1.0) + b
        pltpu.sync_copy(o_v, out_hbm.at[pl.ds(sid * 272, 272)])


def _readout(s3, ids3, bvec):
    return pl.kernel(
        _readout_body,
        out_type=jax.ShapeDtypeStruct((NMP,), _f32),
        mesh=plsc.VectorSubcoreMesh(**_SC_MESH),
        scratch_types=[
            pltpu.VMEM((NGB, GB), jnp.int32),
            pltpu.VMEM((NGB, GB), _f32),
            pltpu.VMEM((GB,), _f32),
            pltpu.VMEM((272,), _f32),
            pltpu.VMEM((272,), _f32),
            pltpu.VMEM((272,), _f32),
            pltpu.VMEM((16,), _f32),
            pltpu.VMEM_SHARED((NMP,), _f32),
            pltpu.VMEM_SHARED((NMP,), _f32),
        ],
    )(s3, ids3, bvec)


def kernel(f_atoms, f_bonds, b2a, b2revb, bond_dst, mol_ids,
           W_i, W_h, W_o, b_o, W_ident, b_ident):
    padb = NBP - NBND
    pada = NAP - NA
    dummy_a = NA + (jnp.arange(padb, dtype=jnp.int32) % pada)
    b2a_p = jnp.concatenate([b2a, dummy_a]).reshape(32, NGB, GB)
    rev_p = jnp.concatenate(
        [b2revb, jnp.arange(NBND, NBP, dtype=jnp.int32)]).reshape(32, NGB, GB)
    dst_p = jnp.concatenate([bond_dst, dummy_a]).reshape(16, NSBB, SBB)
    ids_p = jnp.concatenate(
        [mol_ids, NMOL + (jnp.arange(pada, dtype=jnp.int32) % (NMP - NMOL))]
    ).reshape(16, NGB, GB)

    fb_p = jnp.pad(f_bonds, ((0, padb), (0, BFP - BF)))
    fa_p = jnp.pad(f_atoms, ((0, pada), (0, AFP - AF)))
    wi_p = jnp.pad(W_i, ((0, BFP - BF), (0, HP - HID)))
    wh_p = jnp.pad(W_h, ((0, HP - HID), (0, HP - HID)))
    wo1_p = jnp.pad(W_o[:AF], ((0, AFP - AF), (0, HP - HID)))
    wo2_p = jnp.pad(W_o[AF:], ((0, HP - HID), (0, HP - HID)))
    bo_p = jnp.pad(b_o, (0, HP - HID)).reshape(1, HP)
    wid_p = jnp.pad(W_ident, ((0, HP - HID), (0, 128 - 1)))
    bvec = jnp.full((16,), b_ident[0], _f32)

    inp, zp, zn = _mm_in(fb_p, wi_p, wh_p)
    aw = _scatter(zp, dst_p)
    g1, g2 = _gather(aw, zn, b2a_p, rev_p)
    zp, zn = _mm_update(g1, g2, inp, wh_p)
    aw = _scatter(zp, dst_p)
    g1, g2 = _gather(aw, zn, b2a_p, rev_p)
    y = _mm_update_last(g1, g2, inp, wo2_p)
    ya = _scatter(y, dst_p)
    s = _mm_out(fa_p, ya, wo1_p, bo_p, wid_p)
    s3 = s[:, 0].reshape(16, NGB, GB)
    scores = _readout(s3, ids_p, bvec)
    return scores[:NMOL]


# speedup vs baseline: 1.3009x; 1.3009x over previous
import functools

import jax
import jax.numpy as jnp
from jax import lax
from jax.experimental import pallas as pl
from jax.experimental.pallas import tpu as pltpu
from jax.experimental.pallas import tpu_sc as plsc

NA, NBND, NMOL = 50000, 100000, 4096
AF, BF, HID = 133, 147, 300

HP = 384
BFP = 152
AFP = 136
NAP = 50176
NBP = 100352
PBW = 3136
PBW16 = 6272
NSB16 = 98
GB = 64
NGB = 49
CH = 16
NCH = 24
NCHS = 19
SBB = 112
NSBB = 56
BIG = 784
NBIG = 8
NMP = 4352
APW16 = 3136

_f32 = jnp.float32


def _mm_in_body(fb_ref, wi_ref, wh_ref, inp_ref, zp_ref, zn_ref):
    acc = jnp.dot(fb_ref[...], wi_ref[...], preferred_element_type=_f32)
    inp_ref[...] = acc
    z = jnp.dot(jnp.maximum(acc, 0.0), wh_ref[...], preferred_element_type=_f32)
    zp_ref[...] = z
    zn_ref[...] = -z


def _mm_in(fb_p, wi_p, wh_p):
    blk = 1024
    return pl.pallas_call(
        _mm_in_body,
        grid=(NBP // blk,),
        in_specs=[
            pl.BlockSpec((blk, BF), lambda i: (i, 0)),
            pl.BlockSpec((BF, HP), lambda i: (0, 0)),
            pl.BlockSpec((HP, HP), lambda i: (0, 0)),
        ],
        out_specs=[
            pl.BlockSpec((blk, HP), lambda i: (i, 0)),
            pl.BlockSpec((blk, HP), lambda i: (i, 0)),
            pl.BlockSpec((blk, HP), lambda i: (i, 0)),
        ],
        out_shape=[
            jax.ShapeDtypeStruct((NBP, HP), _f32),
            jax.ShapeDtypeStruct((NBP, HP), _f32),
            jax.ShapeDtypeStruct((NBP, HP), _f32),
        ],
    )(fb_p, wi_p, wh_p)


def _mm_update_body(g1_ref, g2_ref, inp_ref, wh_ref, zp_ref, zn_ref):
    m = jnp.maximum(inp_ref[...] + g1_ref[...] + g2_ref[...], 0.0)
    z = jnp.dot(m, wh_ref[...], preferred_element_type=_f32)
    zp_ref[...] = z
    zn_ref[...] = -z


def _mm_update(g1, g2, inp, wh_p):
    blk = 1024
    return pl.pallas_call(
        _mm_update_body,
        grid=(NBP // blk,),
        in_specs=[
            pl.BlockSpec((blk, HP), lambda i: (i, 0)),
            pl.BlockSpec((blk, HP), lambda i: (i, 0)),
            pl.BlockSpec((blk, HP), lambda i: (i, 0)),
            pl.BlockSpec((HP, HP), lambda i: (0, 0)),
        ],
        out_specs=[
            pl.BlockSpec((blk, HP), lambda i: (i, 0)),
            pl.BlockSpec((blk, HP), lambda i: (i, 0)),
        ],
        out_shape=[
            jax.ShapeDtypeStruct((NBP, HP), _f32),
            jax.ShapeDtypeStruct((NBP, HP), _f32),
        ],
    )(g1, g2, inp, wh_p)


def _mm_update_last_body(g1_ref, g2_ref, inp_ref, wo2_ref, y_ref):
    m = jnp.maximum(inp_ref[...] + g1_ref[...] + g2_ref[...], 0.0)
    y_ref[...] = jnp.dot(m, wo2_ref[...], preferred_element_type=_f32)


def _mm_update_last(g1, g2, inp, wo2_p):
    blk = 1024
    return pl.pallas_call(
        _mm_update_last_body,
        grid=(NBP // blk,),
        in_specs=[
            pl.BlockSpec((blk, HP), lambda i: (i, 0)),
            pl.BlockSpec((blk, HP), lambda i: (i, 0)),
            pl.BlockSpec((blk, HP), lambda i: (i, 0)),
            pl.BlockSpec((HP, HP), lambda i: (0, 0)),
        ],
        out_specs=pl.BlockSpec((blk, HP), lambda i: (i, 0)),
        out_shape=jax.ShapeDtypeStruct((NBP, HP), _f32),
    )(g1, g2, inp, wo2_p)


def _mm_out_body(fa_ref, ya_ref, wo1_ref, bo_ref, wid_ref, s_ref):
    h = (jnp.dot(fa_ref[...], wo1_ref[...], preferred_element_type=_f32)
         + ya_ref[...] + bo_ref[...])
    h = jnp.maximum(h, 0.0)
    s_ref[...] = jnp.dot(h, wid_ref[...], preferred_element_type=_f32)


def _mm_out(fa_p, ya, wo1_p, bo_p, wid_p):
    blk = 1024
    return pl.pallas_call(
        _mm_out_body,
        grid=(NAP // blk,),
        in_specs=[
            pl.BlockSpec((blk, AF), lambda i: (i, 0)),
            pl.BlockSpec((blk, HP), lambda i: (i, 0)),
            pl.BlockSpec((AF, HP), lambda i: (0, 0)),
            pl.BlockSpec((1, HP), lambda i: (0, 0)),
            pl.BlockSpec((HP, 128), lambda i: (0, 0)),
        ],
        out_specs=pl.BlockSpec((blk, 128), lambda i: (i, 0)),
        out_shape=jax.ShapeDtypeStruct((NAP, 128), _f32),
    )(fa_p, ya, wo1_p, bo_p, wid_p)


_SC_MESH = dict(core_axis_name="c", subcore_axis_name="s")


def _scatter_body(zc_hbm, dst_hbm, am_hbm, idx_v, vals_v, zero_v, table,
                  lsem, ssem):
    cid = lax.axis_index("c")
    sid = lax.axis_index("s")
    base = sid * PBW16
    pltpu.sync_copy(dst_hbm.at[sid], idx_v)

    def _zrow(i, carry):
        zero_v[i, :] = jnp.zeros((16,), _f32)
        return carry

    lax.fori_loop(0, 392, _zrow, 0)

    for c2 in range(NCHS, NCH):
        @pl.when(cid == (c2 % 2))
        def _zc(c2=c2):
            for z in range(8):
                pltpu.sync_copy(
                    zero_v, am_hbm.at[c2, pl.ds(sid * 3136 + z * 392, 392), :])

    def _chunk(c, carry):
        ch_id = cid * 10 + c
        for z in range(8):
            pltpu.sync_copy(zero_v, table.at[pl.ds(sid * 3136 + z * 392, 392)])
        plsc.subcore_barrier()
        ld = [None, None]
        pend = [[], []]
        ld[0] = pltpu.async_copy(
            zc_hbm.at[ch_id, pl.ds(base, BIG), :], vals_v.at[0], lsem)
        for b in range(NBIG):
            s = b & 1
            ld[s].wait()
            cur = []
            for k in range(7):
                cur.append(pltpu.async_copy(
                    vals_v.at[s, pl.ds(k * SBB, SBB), :],
                    table.at[idx_v.at[b * 7 + k]], ssem, add=True))
            for d in pend[1 - s]:
                d.wait()
            pend[1 - s] = []
            if b + 1 < NBIG:
                ld[1 - s] = pltpu.async_copy(
                    zc_hbm.at[ch_id, pl.ds(base + (b + 1) * BIG, BIG), :],
                    vals_v.at[1 - s], lsem)
            pend[s] = cur
        for d in pend[0] + pend[1]:
            d.wait()
        plsc.subcore_barrier()
        pltpu.sync_copy(
            table.at[pl.ds(sid * 3136, 3136)],
            am_hbm.at[ch_id, pl.ds(sid * 3136, 3136), :])
        plsc.subcore_barrier()
        return carry

    lax.fori_loop(0, 10 - cid, _chunk, 0)


def _scatter(z, dst3):
    zc = z[:, :NCHS * CH].reshape(NBP, NCHS, CH).swapaxes(0, 1)
    am_c = pl.kernel(
        _scatter_body,
        out_type=jax.ShapeDtypeStruct((NCH, NAP, CH), _f32),
        mesh=plsc.VectorSubcoreMesh(**_SC_MESH),
        scratch_types=[
            pltpu.VMEM((NSBB, SBB), jnp.int32),
            pltpu.VMEM((2, BIG, CH), _f32),
            pltpu.VMEM((392, CH), _f32),
            pltpu.VMEM_SHARED((NAP, CH), _f32),
            pltpu.SemaphoreType.DMA,
            pltpu.SemaphoreType.DMA,
        ],
        compiler_params=pltpu.CompilerParams(use_tc_tiling_on_sc=False),
    )(zc, dst3)
    return am_c.swapaxes(0, 1).reshape(NAP, HP)


def _gather_body(am_hbm, zn_hbm, b2a_hbm, rev_hbm, g1_hbm, g2_hbm,
                 idxa_v, idxr_v, bufa, bufb, sema, semb, semw):
    cid = lax.axis_index("c")
    sid = lax.axis_index("s")
    w = cid * 16 + sid
    base = w * PBW
    pltpu.sync_copy(b2a_hbm.at[w], idxa_v)
    pltpu.sync_copy(rev_hbm.at[w], idxr_v)
    g = [None, None]
    wb = [[], []]
    g[0] = (pltpu.async_copy(am_hbm.at[idxa_v.at[0]], bufa.at[0], sema),
            pltpu.async_copy(zn_hbm.at[idxr_v.at[0]], bufb.at[0], semb))
    for j in range(NGB):
        s = j & 1
        for d in wb[1 - s]:
            d.wait()
        wb[1 - s] = []
        if j + 1 < NGB:
            g[1 - s] = (
                pltpu.async_copy(am_hbm.at[idxa_v.at[j + 1]], bufa.at[1 - s],
                                 sema),
                pltpu.async_copy(zn_hbm.at[idxr_v.at[j + 1]], bufb.at[1 - s],
                                 semb))
        da, db = g[s]
        da.wait()
        db.wait()
        wb[s] = [
            pltpu.async_copy(bufa.at[s], g1_hbm.at[pl.ds(base + j * GB, GB)],
                             semw),
            pltpu.async_copy(bufb.at[s], g2_hbm.at[pl.ds(base + j * GB, GB)],
                             semw),
        ]
    for d in wb[0] + wb[1]:
        d.wait()


def _gather(am, zn, b2a3, rev3):
    return pl.kernel(
        _gather_body,
        out_type=[jax.ShapeDtypeStruct((NBP, HP), _f32),
                  jax.ShapeDtypeStruct((NBP, HP), _f32)],
        mesh=plsc.VectorSubcoreMesh(**_SC_MESH),
        scratch_types=[
            pltpu.VMEM((NGB, GB), jnp.int32),
            pltpu.VMEM((NGB, GB), jnp.int32),
            pltpu.VMEM((2, GB, HP), _f32),
            pltpu.VMEM((2, GB, HP), _f32),
            pltpu.SemaphoreType.DMA,
            pltpu.SemaphoreType.DMA,
            pltpu.SemaphoreType.DMA,
        ],
    )(am, zn, b2a3, rev3)


def _readout_body(s_hbm, ids_hbm, bvec_hbm, out_hbm,
                  idx_v, vals_v, ones_v, a_v, c_v, o_v, bvec_v, sums, counts):
    cid = lax.axis_index("c")
    sid = lax.axis_index("s")

    @pl.when(cid == 0)
    def _():
        pltpu.sync_copy(ids_hbm.at[sid], idx_v)
        pltpu.sync_copy(s_hbm.at[sid], vals_v)
        for k in range(GB // 16):
            ones_v[pl.ds(k * 16, 16)] = jnp.full((16,), 1.0, _f32)
        for k in range(272 // 16):
            o_v[pl.ds(k * 16, 16)] = jnp.zeros((16,), _f32)
        pltpu.sync_copy(o_v, sums.at[pl.ds(sid * 272, 272)])
        pltpu.sync_copy(o_v, counts.at[pl.ds(sid * 272, 272)])
        plsc.subcore_barrier()
        for j in range(NGB):
            pltpu.sync_copy(vals_v.at[j], sums.at[idx_v.at[j]], add=True)
            pltpu.sync_copy(ones_v, counts.at[idx_v.at[j]], add=True)
        plsc.subcore_barrier()
        pltpu.sync_copy(bvec_hbm, bvec_v)
        pltpu.sync_copy(sums.at[pl.ds(sid * 272, 272)], a_v)
        pltpu.sync_copy(counts.at[pl.ds(sid * 272, 272)], c_v)
        b = bvec_v[...]
        for k in range(272 // 16):
            x = a_v[pl.ds(k * 16, 16)]
            cc = c_v[pl.ds(k * 16, 16)]
            o_v[pl.ds(k * 16, 16)] = x / jnp.maximum(cc, 1.0) + b
        pltpu.sync_copy(o_v, out_hbm.at[pl.ds(sid * 272, 272)])


def _readout(s3, ids3, bvec):
    return pl.kernel(
        _readout_body,
        out_type=jax.ShapeDtypeStruct((NMP,), _f32),
        mesh=plsc.VectorSubcoreMesh(**_SC_MESH),
        scratch_types=[
            pltpu.VMEM((NGB, GB), jnp.int32),
            pltpu.VMEM((NGB, GB), _f32),
            pltpu.VMEM((GB,), _f32),
            pltpu.VMEM((272,), _f32),
            pltpu.VMEM((272,), _f32),
            pltpu.VMEM((272,), _f32),
            pltpu.VMEM((16,), _f32),
            pltpu.VMEM_SHARED((NMP,), _f32),
            pltpu.VMEM_SHARED((NMP,), _f32),
        ],
    )(s3, ids3, bvec)


def kernel(f_atoms, f_bonds, b2a, b2revb, bond_dst, mol_ids,
           W_i, W_h, W_o, b_o, W_ident, b_ident):
    padb = NBP - NBND
    pada = NAP - NA
    dummy_a = NA + (jnp.arange(padb, dtype=jnp.int32) % pada)
    b2a_p = jnp.concatenate([b2a, dummy_a]).reshape(32, NGB, GB)
    rev_p = jnp.concatenate(
        [b2revb, jnp.arange(NBND, NBP, dtype=jnp.int32)]).reshape(32, NGB, GB)
    dst_p = jnp.concatenate([bond_dst, dummy_a]).reshape(16, NSBB, SBB)
    ids_p = jnp.concatenate(
        [mol_ids, NMOL + (jnp.arange(pada, dtype=jnp.int32) % (NMP - NMOL))]
    ).reshape(16, NGB, GB)

    wi_p = jnp.pad(W_i, ((0, 0), (0, HP - HID)))
    wh_p = jnp.pad(W_h, ((0, HP - HID), (0, HP - HID)))
    wo1_p = jnp.pad(W_o[:AF], ((0, 0), (0, HP - HID)))
    wo2_p = jnp.pad(W_o[AF:], ((0, HP - HID), (0, HP - HID)))
    bo_p = jnp.pad(b_o, (0, HP - HID)).reshape(1, HP)
    wid_p = jnp.pad(W_ident, ((0, HP - HID), (0, 128 - 1)))
    bvec = jnp.full((16,), b_ident[0], _f32)

    inp, zp, zn = _mm_in(f_bonds, wi_p, wh_p)
    aw = _scatter(zp, dst_p)
    g1, g2 = _gather(aw, zn, b2a_p, rev_p)
    zp, zn = _mm_update(g1, g2, inp, wh_p)
    aw = _scatter(zp, dst_p)
    g1, g2 = _gather(aw, zn, b2a_p, rev_p)
    y = _mm_update_last(g1, g2, inp, wo2_p)
    ya = _scatter(y, dst_p)
    s = _mm_out(f_atoms, ya, wo1_p, bo_p, wid_p)
    s3 = s[:, 0].reshape(16, NGB, GB)
    scores = _readout(s3, ids_p, bvec)
    return scores[:NMOL]


# speedup vs baseline: 1.3166x; 1.0120x over previous
import functools

import jax
import jax.numpy as jnp
from jax import lax
from jax.experimental import pallas as pl
from jax.experimental.pallas import tpu as pltpu
from jax.experimental.pallas import tpu_sc as plsc

NA, NBND, NMOL = 50000, 100000, 4096
AF, BF, HID = 133, 147, 300

HP = 384
BFP = 152
AFP = 136
NAP = 50176
NBP = 100352
PBW = 3136
PBW16 = 6272
NSB16 = 98
GB = 64
NGB = 49
CH = 16
NCH = 24
NCHS = 19
SBB = 112
NSBB = 56
BIG = 784
NBIG = 8
NMP = 4352
APW16 = 3136

_f32 = jnp.float32


def _mm_in_body(fb_ref, wi_ref, wh_ref, inp_ref, zp_ref):
    acc = jnp.dot(fb_ref[...], wi_ref[...], preferred_element_type=_f32)
    inp_ref[...] = acc
    zp_ref[...] = jnp.dot(jnp.maximum(acc, 0.0), wh_ref[...],
                          preferred_element_type=_f32)


def _mm_in(fb_p, wi_p, wh_p):
    blk = 1024
    return pl.pallas_call(
        _mm_in_body,
        grid=(NBP // blk,),
        in_specs=[
            pl.BlockSpec((blk, BF), lambda i: (i, 0)),
            pl.BlockSpec((BF, HP), lambda i: (0, 0)),
            pl.BlockSpec((HP, HP), lambda i: (0, 0)),
        ],
        out_specs=[
            pl.BlockSpec((blk, HP), lambda i: (i, 0)),
            pl.BlockSpec((blk, HP), lambda i: (i, 0)),
        ],
        out_shape=[
            jax.ShapeDtypeStruct((NBP, HP), _f32),
            jax.ShapeDtypeStruct((NBP, HP), _f32),
        ],
    )(fb_p, wi_p, wh_p)


def _mm_update_body(g1_ref, g2_ref, inp_ref, wh_ref, zp_ref):
    m = jnp.maximum(inp_ref[...] + g1_ref[...] - g2_ref[...], 0.0)
    zp_ref[...] = jnp.dot(m, wh_ref[...], preferred_element_type=_f32)


def _mm_update(g1, g2, inp, wh_p):
    blk = 1024
    return pl.pallas_call(
        _mm_update_body,
        grid=(NBP // blk,),
        in_specs=[
            pl.BlockSpec((blk, HP), lambda i: (i, 0)),
            pl.BlockSpec((blk, HP), lambda i: (i, 0)),
            pl.BlockSpec((blk, HP), lambda i: (i, 0)),
            pl.BlockSpec((HP, HP), lambda i: (0, 0)),
        ],
        out_specs=pl.BlockSpec((blk, HP), lambda i: (i, 0)),
        out_shape=jax.ShapeDtypeStruct((NBP, HP), _f32),
    )(g1, g2, inp, wh_p)


def _mm_update_last_body(g1_ref, g2_ref, inp_ref, wo2_ref, y_ref):
    m = jnp.maximum(inp_ref[...] + g1_ref[...] - g2_ref[...], 0.0)
    y_ref[...] = jnp.dot(m, wo2_ref[...], preferred_element_type=_f32)


def _mm_update_last(g1, g2, inp, wo2_p):
    blk = 1024
    return pl.pallas_call(
        _mm_update_last_body,
        grid=(NBP // blk,),
        in_specs=[
            pl.BlockSpec((blk, HP), lambda i: (i, 0)),
            pl.BlockSpec((blk, HP), lambda i: (i, 0)),
            pl.BlockSpec((blk, HP), lambda i: (i, 0)),
            pl.BlockSpec((HP, HP), lambda i: (0, 0)),
        ],
        out_specs=pl.BlockSpec((blk, HP), lambda i: (i, 0)),
        out_shape=jax.ShapeDtypeStruct((NBP, HP), _f32),
    )(g1, g2, inp, wo2_p)


def _mm_out_body(fa_ref, ya_ref, wo1_ref, bo_ref, wid_ref, s_ref):
    h = (jnp.dot(fa_ref[...], wo1_ref[...], preferred_element_type=_f32)
         + ya_ref[...] + bo_ref[...])
    h = jnp.maximum(h, 0.0)
    s_ref[...] = jnp.dot(h, wid_ref[...], preferred_element_type=_f32)


def _mm_out(fa_p, ya, wo1_p, bo_p, wid_p):
    blk = 1024
    return pl.pallas_call(
        _mm_out_body,
        grid=(NAP // blk,),
        in_specs=[
            pl.BlockSpec((blk, AF), lambda i: (i, 0)),
            pl.BlockSpec((blk, HP), lambda i: (i, 0)),
            pl.BlockSpec((AF, HP), lambda i: (0, 0)),
            pl.BlockSpec((1, HP), lambda i: (0, 0)),
            pl.BlockSpec((HP, 128), lambda i: (0, 0)),
        ],
        out_specs=pl.BlockSpec((blk, 128), lambda i: (i, 0)),
        out_shape=jax.ShapeDtypeStruct((NAP, 128), _f32),
    )(fa_p, ya, wo1_p, bo_p, wid_p)


_SC_MESH = dict(core_axis_name="c", subcore_axis_name="s")


def _scatter_body(zc_hbm, dst_hbm, am_hbm, idx_v, vals_v, zero_v, table,
                  lsem, ssem):
    cid = lax.axis_index("c")
    sid = lax.axis_index("s")
    base = sid * PBW16
    pltpu.sync_copy(dst_hbm.at[sid], idx_v)

    def _zrow(i, carry):
        zero_v[i, :] = jnp.zeros((16,), _f32)
        return carry

    lax.fori_loop(0, 392, _zrow, 0)

    for c2 in range(NCHS, NCH):
        @pl.when(cid == (c2 % 2))
        def _zc(c2=c2):
            for z in range(8):
                pltpu.sync_copy(
                    zero_v, am_hbm.at[c2, pl.ds(sid * 3136 + z * 392, 392), :])

    def _chunk(c, carry):
        ch_id = cid * 10 + c
        for z in range(8):
            pltpu.sync_copy(zero_v, table.at[pl.ds(sid * 3136 + z * 392, 392)])
        plsc.subcore_barrier()
        ld = [None, None]
        pend = [[], []]
        ld[0] = pltpu.async_copy(
            zc_hbm.at[ch_id, pl.ds(base, BIG), :], vals_v.at[0], lsem)
        for b in range(NBIG):
            s = b & 1
            ld[s].wait()
            cur = []
            for k in range(7):
                cur.append(pltpu.async_copy(
                    vals_v.at[s, pl.ds(k * SBB, SBB), :],
                    table.at[idx_v.at[b * 7 + k]], ssem, add=True))
            for d in pend[1 - s]:
                d.wait()
            pend[1 - s] = []
            if b + 1 < NBIG:
                ld[1 - s] = pltpu.async_copy(
                    zc_hbm.at[ch_id, pl.ds(base + (b + 1) * BIG, BIG), :],
                    vals_v.at[1 - s], lsem)
            pend[s] = cur
        for d in pend[0] + pend[1]:
            d.wait()
        plsc.subcore_barrier()
        pltpu.sync_copy(
            table.at[pl.ds(sid * 3136, 3136)],
            am_hbm.at[ch_id, pl.ds(sid * 3136, 3136), :])
        plsc.subcore_barrier()
        return carry

    lax.fori_loop(0, 10 - cid, _chunk, 0)


def _scatter(z, dst3):
    zc = z[:, :NCHS * CH].reshape(NBP, NCHS, CH).swapaxes(0, 1)
    am_c = pl.kernel(
        _scatter_body,
        out_type=jax.ShapeDtypeStruct((NCH, NAP, CH), _f32),
        mesh=plsc.VectorSubcoreMesh(**_SC_MESH),
        scratch_types=[
            pltpu.VMEM((NSBB, SBB), jnp.int32),
            pltpu.VMEM((2, BIG, CH), _f32),
            pltpu.VMEM((392, CH), _f32),
            pltpu.VMEM_SHARED((NAP, CH), _f32),
            pltpu.SemaphoreType.DMA,
            pltpu.SemaphoreType.DMA,
        ],
        compiler_params=pltpu.CompilerParams(use_tc_tiling_on_sc=False),
    )(zc, dst3)
    return am_c.swapaxes(0, 1).reshape(NAP, HP)


def _gather_body(am_hbm, zn_hbm, b2a_hbm, rev_hbm, g1_hbm, g2_hbm,
                 idxa_v, idxr_v, bufa, bufb, sema, semb, semw):
    cid = lax.axis_index("c")
    sid = lax.axis_index("s")
    w = cid * 16 + sid
    base = w * PBW
    pltpu.sync_copy(b2a_hbm.at[w], idxa_v)
    pltpu.sync_copy(rev_hbm.at[w], idxr_v)
    g = [None, None]
    wb = [[], []]
    g[0] = (pltpu.async_copy(am_hbm.at[idxa_v.at[0]], bufa.at[0], sema),
            pltpu.async_copy(zn_hbm.at[idxr_v.at[0]], bufb.at[0], semb))
    for j in range(NGB):
        s = j & 1
        for d in wb[1 - s]:
            d.wait()
        wb[1 - s] = []
        if j + 1 < NGB:
            g[1 - s] = (
                pltpu.async_copy(am_hbm.at[idxa_v.at[j + 1]], bufa.at[1 - s],
                                 sema),
                pltpu.async_copy(zn_hbm.at[idxr_v.at[j + 1]], bufb.at[1 - s],
                                 semb))
        da, db = g[s]
        da.wait()
        db.wait()
        wb[s] = [
            pltpu.async_copy(bufa.at[s], g1_hbm.at[pl.ds(base + j * GB, GB)],
                             semw),
            pltpu.async_copy(bufb.at[s], g2_hbm.at[pl.ds(base + j * GB, GB)],
                             semw),
        ]
    for d in wb[0] + wb[1]:
        d.wait()


def _gather(am, zn, b2a3, rev3):
    return pl.kernel(
        _gather_body,
        out_type=[jax.ShapeDtypeStruct((NBP, HP), _f32),
                  jax.ShapeDtypeStruct((NBP, HP), _f32)],
        mesh=plsc.VectorSubcoreMesh(**_SC_MESH),
        scratch_types=[
            pltpu.VMEM((NGB, GB), jnp.int32),
            pltpu.VMEM((NGB, GB), jnp.int32),
            pltpu.VMEM((2, GB, HP), _f32),
            pltpu.VMEM((2, GB, HP), _f32),
            pltpu.SemaphoreType.DMA,
            pltpu.SemaphoreType.DMA,
            pltpu.SemaphoreType.DMA,
        ],
    )(am, zn, b2a3, rev3)


def _readout_body(s_hbm, ids_hbm, bvec_hbm, out_hbm,
                  idx_v, vals_v, ones_v, a_v, c_v, o_v, bvec_v, sums, counts):
    cid = lax.axis_index("c")
    sid = lax.axis_index("s")

    @pl.when(cid == 0)
    def _():
        pltpu.sync_copy(ids_hbm.at[sid], idx_v)
        pltpu.sync_copy(s_hbm.at[sid], vals_v)
        for k in range(GB // 16):
            ones_v[pl.ds(k * 16, 16)] = jnp.full((16,), 1.0, _f32)
        for k in range(272 // 16):
            o_v[pl.ds(k * 16, 16)] = jnp.zeros((16,), _f32)
        pltpu.sync_copy(o_v, sums.at[pl.ds(sid * 272, 272)])
        pltpu.sync_copy(o_v, counts.at[pl.ds(sid * 272, 272)])
        plsc.subcore_barrier()
        for j in range(NGB):
            pltpu.sync_copy(vals_v.at[j], sums.at[idx_v.at[j]], add=True)
            pltpu.sync_copy(ones_v, counts.at[idx_v.at[j]], add=True)
        plsc.subcore_barrier()
        pltpu.sync_copy(bvec_hbm, bvec_v)
        pltpu.sync_copy(sums.at[pl.ds(sid * 272, 272)], a_v)
        pltpu.sync_copy(counts.at[pl.ds(sid * 272, 272)], c_v)
        b = bvec_v[...]
        for k in range(272 // 16):
            x = a_v[pl.ds(k * 16, 16)]
            cc = c_v[pl.ds(k * 16, 16)]
            o_v[pl.ds(k * 16, 16)] = x / jnp.maximum(cc, 1.0) + b
        pltpu.sync_copy(o_v, out_hbm.at[pl.ds(sid * 272, 272)])


def _readout(s3, ids3, bvec):
    return pl.kernel(
        _readout_body,
        out_type=jax.ShapeDtypeStruct((NMP,), _f32),
        mesh=plsc.VectorSubcoreMesh(**_SC_MESH),
        scratch_types=[
            pltpu.VMEM((NGB, GB), jnp.int32),
            pltpu.VMEM((NGB, GB), _f32),
            pltpu.VMEM((GB,), _f32),
            pltpu.VMEM((272,), _f32),
            pltpu.VMEM((272,), _f32),
            pltpu.VMEM((272,), _f32),
            pltpu.VMEM((16,), _f32),
            pltpu.VMEM_SHARED((NMP,), _f32),
            pltpu.VMEM_SHARED((NMP,), _f32),
        ],
    )(s3, ids3, bvec)


def kernel(f_atoms, f_bonds, b2a, b2revb, bond_dst, mol_ids,
           W_i, W_h, W_o, b_o, W_ident, b_ident):
    padb = NBP - NBND
    pada = NAP - NA
    dummy_a = NA + (jnp.arange(padb, dtype=jnp.int32) % pada)
    b2a_p = jnp.concatenate([b2a, dummy_a]).reshape(32, NGB, GB)
    rev_p = jnp.concatenate(
        [b2revb, jnp.arange(NBND, NBP, dtype=jnp.int32)]).reshape(32, NGB, GB)
    dst_p = jnp.concatenate([bond_dst, dummy_a]).reshape(16, NSBB, SBB)
    ids_p = jnp.concatenate(
        [mol_ids, NMOL + (jnp.arange(pada, dtype=jnp.int32) % (NMP - NMOL))]
    ).reshape(16, NGB, GB)

    wi_p = jnp.pad(W_i, ((0, 0), (0, HP - HID)))
    wh_p = jnp.pad(W_h, ((0, HP - HID), (0, HP - HID)))
    wo1_p = jnp.pad(W_o[:AF], ((0, 0), (0, HP - HID)))
    wo2_p = jnp.pad(W_o[AF:], ((0, HP - HID), (0, HP - HID)))
    bo_p = jnp.pad(b_o, (0, HP - HID)).reshape(1, HP)
    wid_p = jnp.pad(W_ident, ((0, HP - HID), (0, 128 - 1)))
    bvec = jnp.full((16,), b_ident[0], _f32)

    inp, zp = _mm_in(f_bonds, wi_p, wh_p)
    aw = _scatter(zp, dst_p)
    g1, g2 = _gather(aw, zp, b2a_p, rev_p)
    zp = _mm_update(g1, g2, inp, wh_p)
    aw = _scatter(zp, dst_p)
    g1, g2 = _gather(aw, zp, b2a_p, rev_p)
    y = _mm_update_last(g1, g2, inp, wo2_p)
    ya = _scatter(y, dst_p)
    s = _mm_out(f_atoms, ya, wo1_p, bo_p, wid_p)
    s3 = s[:, 0].reshape(16, NGB, GB)
    scores = _readout(s3, ids_p, bvec)
    return scores[:NMOL]
